# baseline (speedup 1.0000x reference)
import jax
import jax.numpy as jnp
from jax.experimental import pallas as pl
from jax.experimental.pallas import tpu as pltpu

N = 10000
SVD = 128
H = 4
D = 16
HD = H * D
EMB = 64
M = 2
NEG_SLOPE = 0.1

TM = 400


def _emb_body(acc_ref, w_ref, b_ref, emb_ref):
    emb_ref[...] = jnp.dot(acc_ref[...], w_ref[...],
                           preferred_element_type=jnp.float32) + b_ref[...]


def _dot_body(a_ref, b_ref, o_ref):
    o_ref[...] = jax.lax.dot_general(
        a_ref[...], b_ref[...], (((1,), (1,)), ((), ())),
        preferred_element_type=jnp.float32)


def _emb_dot(acc, emb_W, emb_b):
    emb = pl.pallas_call(
        _emb_body,
        out_shape=jax.ShapeDtypeStruct((N, EMB), jnp.float32),
    )(acc, emb_W, emb_b.reshape(1, EMB))
    dot = pl.pallas_call(
        _dot_body,
        grid=(N // TM,),
        in_specs=[
            pl.BlockSpec((TM, EMB), lambda i: (i, 0)),
            pl.BlockSpec((N, EMB), lambda i: (0, 0)),
        ],
        out_specs=pl.BlockSpec((TM, N), lambda i: (i, 0)),
        out_shape=jax.ShapeDtypeStruct((N, N), jnp.float32),
    )(emb, emb)
    return dot, emb


def _wgat(x, edge_index, edge_weight, lin_src_W, lin_dst_W, att_src, att_dst, bias):
    src = edge_index[0]
    dst = edge_index[1]
    loops = jnp.arange(N)
    src = jnp.concatenate([src, loops])
    dst = jnp.concatenate([dst, loops])
    ew = jnp.concatenate([edge_weight, jnp.ones((N,), dtype=edge_weight.dtype)])
    xs = (x @ lin_src_W).reshape(N, H, D)
    xd = (x @ lin_dst_W).reshape(N, H, D)
    a_src = (xs * att_src[None, :, :]).sum(-1)
    a_dst = (xd * att_dst[None, :, :]).sum(-1)
    alpha = a_src[src] + a_dst[dst]
    alpha = jax.nn.leaky_relu(alpha, NEG_SLOPE)
    amax = jax.ops.segment_max(alpha, dst, num_segments=N)
    amax = jnp.where(jnp.isfinite(amax), amax, 0.0)
    ex = jnp.exp(alpha - amax[dst])
    denom = jax.ops.segment_sum(ex, dst, num_segments=N)
    alpha = ex / (denom[dst] + 1e-16)
    alpha = alpha * ew[:, None]
    msg = xs[src] * alpha[:, :, None]
    out = jax.ops.segment_sum(msg, dst, num_segments=N)
    return out.reshape(N, HD) + bias


def kernel(features, masks, edge_index0, edge_weight0, edge_index1, edge_weight1,
           pre_W0, pre_b0, pre_W1, pre_b1,
           lin_src_W0, lin_dst_W0, att_src0, att_dst0, gat_b0,
           lin_src_W1, lin_dst_W1, att_src1, att_dst1, gat_b1,
           scales, emb_W, emb_b):
    net_scales = jax.nn.softmax(scales.reshape(1, -1), axis=-1)
    interp_masks = jax.nn.softmax(masks, axis=-1)
    pre = [(pre_W0, pre_b0), (pre_W1, pre_b1)]
    gat = [(lin_src_W0, lin_dst_W0, att_src0, att_dst0, gat_b0),
           (lin_src_W1, lin_dst_W1, att_src1, att_dst1, gat_b1)]
    edges = [(edge_index0, edge_weight0), (edge_index1, edge_weight1)]
    acc = jnp.zeros((N, HD), dtype=jnp.float32)
    for i in range(M):
        x = features @ pre[i][0] + pre[i][1]
        x = _wgat(x, edges[i][0], edges[i][1], *gat[i])
        x = net_scales[0, i] * interp_masks[:, i].reshape(-1, 1) * x
        acc = acc + x
    dot, emb = _emb_dot(acc, emb_W, emb_b)
    return dot, emb, net_scales


# speedup vs baseline: 41.4902x; 41.4902x over previous
import functools

import jax
import jax.numpy as jnp
from jax import lax
from jax.experimental import pallas as pl
from jax.experimental.pallas import tpu as pltpu
from jax.experimental.pallas import tpu_sc as plsc

N = 10000
SVD = 128
H = 4
D = 16
HD = H * D
EMB = 64
M = 2
NEG = 0.1

NTILE = 16
K = 128
E = 640000
EPT = 40960
E_PAD = NTILE * EPT
NCHUNK = EPT // K
NPAD = N + 8
ACCW = 72
SR = 200
NSLICE = N // SR

TM = 400
TR = 2000


def _prep_body(feat_ref, preW_ref, preb_ref, linS_ref, linD_ref,
               AS_ref, AD_ref, xs_ref, asrc_ref, dpk_ref):
    f = feat_ref[...]
    for i in range(M):
        x = jnp.dot(f, preW_ref[i], preferred_element_type=jnp.float32)
        x = x + preb_ref[i]
        xs = jnp.dot(x, linS_ref[i], preferred_element_type=jnp.float32)
        xd = jnp.dot(x, linD_ref[i], preferred_element_type=jnp.float32)
        a_s = jnp.dot(xs, AS_ref[i], preferred_element_type=jnp.float32)
        a_d = jnp.dot(xd, AD_ref[i], preferred_element_type=jnp.float32)
        g = jnp.max(a_s, axis=0, keepdims=True)
        z = g + a_d
        ub = jnp.maximum(z, NEG * z)
        xs_ref[i, ...] = xs
        asrc_ref[i, ...] = a_s
        dpk_ref[i, :, 0:4] = a_d
        dpk_ref[i, :, 4:8] = ub


def _prep(features, preW, preb, linS, linD, AS, AD):
    return pl.pallas_call(
        _prep_body,
        out_shape=(
            jax.ShapeDtypeStruct((M, N, HD), jnp.float32),
            jax.ShapeDtypeStruct((M, N, H), jnp.float32),
            jax.ShapeDtypeStruct((M, N, 2 * H), jnp.float32),
        ),
    )(features, preW, preb, linS, linD, AS, AD)


_mesh = plsc.VectorSubcoreMesh(core_axis_name="c", subcore_axis_name="s")


@functools.partial(
    pl.kernel,
    out_type=jax.ShapeDtypeStruct((M, N, ACCW), jnp.float32),
    mesh=_mesh,
    compiler_params=pltpu.CompilerParams(needs_layout_passes=False,
                                         use_tc_tiling_on_sc=False),
    scratch_types=[
        pltpu.VMEM_SHARED((NPAD, ACCW), jnp.float32),
        pltpu.VMEM_SHARED((2 * N, 2 * H), jnp.float32),
        pltpu.VMEM_SHARED((NPAD, 2 * H), jnp.float32),
        pltpu.VMEM((K,), jnp.int32),
        pltpu.VMEM((K,), jnp.int32),
        pltpu.VMEM((K,), jnp.float32),
        pltpu.VMEM((K, HD), jnp.float32),
        pltpu.VMEM((K, 2 * H), jnp.float32),
        pltpu.VMEM((K, 2 * H), jnp.float32),
        pltpu.VMEM((K, H), jnp.float32),
        pltpu.VMEM((K, ACCW), jnp.float32),
        pltpu.VMEM((SR, ACCW), jnp.float32),
        pltpu.VMEM((SR, 2 * H), jnp.float32),
        pltpu.VMEM((SR, 2 * H), jnp.float32),
        pltpu.VMEM((8, 2 * H), jnp.float32),
    ],
)
def _sc_edge(src_hbm, dst_hbm, ew_hbm, xs_hbm, asrc_hbm, dpk_hbm, zrow_hbm,
             z8_hbm, out_hbm, acc_sh, asrc_sh, dpk_sh, srcv, dstv, ewv,
             xsb, asb, dpb, wb, sb, tmp, stg4, stg8, z8v):
    c = lax.axis_index("c")
    t = lax.axis_index("s")

    pltpu.sync_copy(zrow_hbm, tmp)
    for qi in range(2 * NSLICE // NTILE + 1):
        q = t + qi * NTILE

        @pl.when(q < 2 * NSLICE)
        def _():
            r = q * SR
            pltpu.sync_copy(asrc_hbm.at[pl.ds(r, SR)], stg4)
            pltpu.sync_copy(stg4, asrc_sh.at[pl.ds(r, SR)])

    for qi in range(4):
        q = t + qi * NTILE

        @pl.when(q < NSLICE)
        def _():
            r = q * SR
            pltpu.sync_copy(dpk_hbm.at[c, pl.ds(r, SR)], stg8)
            pltpu.sync_copy(stg8, dpk_sh.at[pl.ds(r, SR)])
            pltpu.sync_copy(tmp, acc_sh.at[pl.ds(r, SR)])

    @pl.when(t == 0)
    def _():
        pltpu.sync_copy(tmp.at[pl.ds(0, 8)], acc_sh.at[pl.ds(N, 8)])
        pltpu.sync_copy(z8_hbm, z8v)
        pltpu.sync_copy(z8v, dpk_sh.at[pl.ds(N, 8)])

    lanes = lax.iota(jnp.int32, 16)
    zv = jnp.zeros((16,), jnp.float32)
    for g in range(K // 16):
        for pc in range(HD + H, ACCW):
            plsc.store_scatter(sb, [lanes + g * 16,
                                    jnp.full((16,), pc, jnp.int32)], zv)

    plsc.subcore_barrier()

    quad = lanes // 4
    hmod = lanes - quad * 4

    def chunk_body(ci, carry):
        base = c * E_PAD + t * EPT + ci * K
        pltpu.sync_copy(src_hbm.at[pl.ds(base, K)], srcv)
        pltpu.sync_copy(dst_hbm.at[pl.ds(base, K)], dstv)
        pltpu.sync_copy(ew_hbm.at[pl.ds(base, K)], ewv)
        pltpu.sync_copy(xs_hbm.at[srcv], xsb)
        pltpu.sync_copy(asrc_sh.at[srcv], asb)
        pltpu.sync_copy(dpk_sh.at[dstv], dpb)

        for j in range(K * H // 16):
            ev = quad + j * 4
            a_s = plsc.load_gather(asb, [ev, hmod])
            a_d = plsc.load_gather(dpb, [ev, hmod])
            ubv = plsc.load_gather(dpb, [ev, hmod + H])
            eww = plsc.load_gather(ewv, [ev])
            a = a_s + a_d
            a = jnp.maximum(a, NEG * a)
            ex = jnp.exp(a - ubv)
            plsc.store_scatter(sb, [ev, hmod + HD], ex)
            plsc.store_scatter(wb, [ev, hmod], ex * eww)

        def grp(g, carry2):
            evg = lanes + g * 16
            for h in range(H):
                wv = plsc.load_gather(wb, [evg, jnp.full((16,), h, jnp.int32)])
                for d in range(D):
                    col = jnp.full((16,), h * D + d, jnp.int32)
                    xv = plsc.load_gather(xsb, [evg, col])
                    plsc.store_scatter(sb, [evg, col], xv * wv)
            return carry2

        lax.fori_loop(0, K // 16, grp, 0)

        pltpu.sync_copy(sb, acc_sh.at[dstv], add=True)
        return carry

    lax.fori_loop(0, NCHUNK, chunk_body, 0)

    plsc.subcore_barrier()

    for qi in range(4):
        q = t + qi * NTILE

        @pl.when(q < NSLICE)
        def _():
            r = q * SR
            pltpu.sync_copy(acc_sh.at[pl.ds(r, SR)], tmp)
            pltpu.sync_copy(tmp, out_hbm.at[c, pl.ds(r, SR)])


def _final_body(acc_ref, xs_ref, asrc_ref, dpk_ref, gatb_ref, masks_ref,
                scales_ref, embW_ref, embb_ref, R_ref, emb_ref, ns_ref):
    ns = jax.nn.softmax(scales_ref[...], axis=-1)
    im = jax.nn.softmax(masks_ref[...], axis=-1)
    R = R_ref[...]
    accn = jnp.zeros((TR, HD), jnp.float32)
    for i in range(M):
        msg = acc_ref[i, :, 0:HD]
        dsum = acc_ref[i, :, HD:HD + H]
        s = asrc_ref[i] + dpk_ref[i, :, 0:H]
        ub = dpk_ref[i, :, H:2 * H]
        asl = jnp.exp(jnp.maximum(s, NEG * s) - ub)
        rec = 1.0 / (dsum + asl + 1e-16)
        asl64 = jnp.dot(asl, R, preferred_element_type=jnp.float32)
        rec64 = jnp.dot(rec, R, preferred_element_type=jnp.float32)
        conv = (msg + asl64 * xs_ref[i]) * rec64 + gatb_ref[i]
        accn = accn + conv * (im[:, i:i + 1] * ns[:, i:i + 1])
    emb_ref[...] = jnp.dot(accn, embW_ref[...],
                           preferred_element_type=jnp.float32) + embb_ref[...]
    ns_ref[...] = ns


def _final(acc, xs_all, asrc_all, dpk_all, gatb, masks, scales, embW, embb, R):
    return pl.pallas_call(
        _final_body,
        grid=(N // TR,),
        in_specs=[
            pl.BlockSpec((M, TR, ACCW), lambda r: (0, r, 0)),
            pl.BlockSpec((M, TR, HD), lambda r: (0, r, 0)),
            pl.BlockSpec((M, TR, H), lambda r: (0, r, 0)),
            pl.BlockSpec((M, TR, 2 * H), lambda r: (0, r, 0)),
            pl.BlockSpec((M, 1, HD), lambda r: (0, 0, 0)),
            pl.BlockSpec((TR, M), lambda r: (r, 0)),
            pl.BlockSpec((1, M), lambda r: (0, 0)),
            pl.BlockSpec((HD, EMB), lambda r: (0, 0)),
            pl.BlockSpec((1, EMB), lambda r: (0, 0)),
            pl.BlockSpec((H, HD), lambda r: (0, 0)),
        ],
        out_specs=(
            pl.BlockSpec((TR, EMB), lambda r: (r, 0)),
            pl.BlockSpec((1, M), lambda r: (0, 0)),
        ),
        out_shape=(
            jax.ShapeDtypeStruct((N, EMB), jnp.float32),
            jax.ShapeDtypeStruct((1, M), jnp.float32),
        ),
    )(acc, xs_all, asrc_all, dpk_all, gatb, masks, scales, embW, embb, R)


def _dot_body(a_ref, b_ref, o_ref):
    o_ref[...] = lax.dot_general(
        a_ref[...], b_ref[...], (((1,), (1,)), ((), ())),
        preferred_element_type=jnp.float32)


def _dot(emb):
    return pl.pallas_call(
        _dot_body,
        grid=(N // TM,),
        in_specs=[
            pl.BlockSpec((TM, EMB), lambda i: (i, 0)),
            pl.BlockSpec((N, EMB), lambda i: (0, 0)),
        ],
        out_specs=pl.BlockSpec((TM, N), lambda i: (i, 0)),
        out_shape=jax.ShapeDtypeStruct((N, N), jnp.float32),
    )(emb, emb)


def kernel(features, masks, edge_index0, edge_weight0, edge_index1, edge_weight1,
           pre_W0, pre_b0, pre_W1, pre_b1,
           lin_src_W0, lin_dst_W0, att_src0, att_dst0, gat_b0,
           lin_src_W1, lin_dst_W1, att_src1, att_dst1, gat_b1,
           scales, emb_W, emb_b):
    f32 = jnp.float32
    i32 = jnp.int32
    preW = jnp.stack([pre_W0, pre_W1])
    preb = jnp.stack([pre_b0.reshape(1, HD), pre_b1.reshape(1, HD)])
    linS = jnp.stack([lin_src_W0, lin_src_W1])
    linD = jnp.stack([lin_dst_W0, lin_dst_W1])
    R = jnp.kron(jnp.eye(H, dtype=f32), jnp.ones((1, D), f32))
    AS = jnp.stack([R.T * att_src0.reshape(HD, 1),
                    R.T * att_src1.reshape(HD, 1)])
    AD = jnp.stack([R.T * att_dst0.reshape(HD, 1),
                    R.T * att_dst1.reshape(HD, 1)])
    gatb = jnp.stack([gat_b0.reshape(1, HD), gat_b1.reshape(1, HD)])

    xs_all, asrc_all, dpk_all = _prep(features, preW, preb, linS, linD, AS, AD)

    pad = E_PAD - E
    src_flat = jnp.concatenate([
        edge_index0[0], jnp.zeros((pad,), i32),
        edge_index1[0] + N, jnp.full((pad,), N, i32)])
    dst_flat = jnp.concatenate([
        edge_index0[1], jnp.full((pad,), N, i32),
        edge_index1[1], jnp.full((pad,), N, i32)])
    ew_flat = jnp.concatenate([
        edge_weight0, jnp.zeros((pad,), f32),
        edge_weight1, jnp.zeros((pad,), f32)])
    zrow = jnp.zeros((SR, ACCW), f32)
    z8 = jnp.zeros((8, 2 * H), f32)

    asrc8 = jnp.concatenate(
        [asrc_all.reshape(M * N, H), jnp.zeros((M * N, H), f32)], axis=1)
    acc = _sc_edge(src_flat, dst_flat, ew_flat, xs_all.reshape(M * N, HD),
                   asrc8, dpk_all, zrow, z8)

    emb, ns = _final(acc, xs_all, asrc_all, dpk_all, gatb, masks,
                     scales.reshape(1, M), emb_W, emb_b.reshape(1, EMB), R)
    dot = _dot(emb)
    return dot, emb, ns


# speedup vs baseline: 47.3757x; 1.1419x over previous
import functools

import jax
import jax.numpy as jnp
from jax import lax
from jax.experimental import pallas as pl
from jax.experimental.pallas import tpu as pltpu
from jax.experimental.pallas import tpu_sc as plsc

N = 10000
SVD = 128
H = 4
D = 16
HD = H * D
EMB = 64
M = 2
NEG = 0.1

NTILE = 16
K = 128
E = 640000
EPT = 40960
E_PAD = NTILE * EPT
NCHUNK = EPT // K
NBLK = 16
NBLOCK = NCHUNK // NBLK
NPAD = N + 8
ACCW = 72
SR = 200
NSLICE = N // SR

TM = 400
TR = 2000


def _prep_body(feat_ref, preW_ref, preb_ref, linS_ref, linD_ref,
               AS_ref, AD_ref, xs_ref, asrc_ref, dpk_ref):
    f = feat_ref[...]
    for i in range(M):
        x = jnp.dot(f, preW_ref[i], preferred_element_type=jnp.float32)
        x = x + preb_ref[i]
        xs = jnp.dot(x, linS_ref[i], preferred_element_type=jnp.float32)
        xd = jnp.dot(x, linD_ref[i], preferred_element_type=jnp.float32)
        a_s = jnp.dot(xs, AS_ref[i], preferred_element_type=jnp.float32)
        a_d = jnp.dot(xd, AD_ref[i], preferred_element_type=jnp.float32)
        g = jnp.max(a_s, axis=0, keepdims=True)
        z = g + a_d
        ub = jnp.maximum(z, NEG * z)
        xs_ref[i, ...] = xs
        asrc_ref[i, ...] = a_s
        dpk_ref[i, :, 0:4] = a_d
        dpk_ref[i, :, 4:8] = ub


def _prep(features, preW, preb, linS, linD, AS, AD):
    return pl.pallas_call(
        _prep_body,
        out_shape=(
            jax.ShapeDtypeStruct((M, N, HD), jnp.float32),
            jax.ShapeDtypeStruct((M, N, H), jnp.float32),
            jax.ShapeDtypeStruct((M, N, 2 * H), jnp.float32),
        ),
    )(features, preW, preb, linS, linD, AS, AD)


_mesh = plsc.VectorSubcoreMesh(core_axis_name="c", subcore_axis_name="s")


@functools.partial(
    pl.kernel,
    out_type=jax.ShapeDtypeStruct((M, N, ACCW), jnp.float32),
    mesh=_mesh,
    compiler_params=pltpu.CompilerParams(needs_layout_passes=False,
                                         use_tc_tiling_on_sc=False),
    scratch_types=[
        pltpu.VMEM_SHARED((NPAD, ACCW), jnp.float32),
        pltpu.VMEM_SHARED((2 * N, 2 * H), jnp.float32),
        pltpu.VMEM_SHARED((NPAD, 2 * H), jnp.float32),
        pltpu.VMEM((NBLK, K), jnp.int32),
        pltpu.VMEM((NBLK, K), jnp.int32),
        pltpu.VMEM((NBLK, K), jnp.float32),
        pltpu.VMEM((K, HD), jnp.float32),
        pltpu.VMEM((K, 2 * H), jnp.float32),
        pltpu.VMEM((K, 2 * H), jnp.float32),
        pltpu.VMEM((K, H), jnp.float32),
        pltpu.VMEM((K, ACCW), jnp.float32),
        pltpu.VMEM((SR, ACCW), jnp.float32),
        pltpu.VMEM((SR, 2 * H), jnp.float32),
        pltpu.VMEM((SR, 2 * H), jnp.float32),
        pltpu.VMEM((8, 2 * H), jnp.float32),
        pltpu.SemaphoreType.DMA,
        pltpu.SemaphoreType.DMA,
        pltpu.SemaphoreType.DMA,
    ],
)
def _sc_edge(src_hbm, dst_hbm, ew_hbm, xs_hbm, asrc_hbm, dpk_hbm, zrow_hbm,
             z8_hbm, out_hbm, acc_sh, asrc_sh, dpk_sh, srcb, dstb, ewb,
             xsb, asb, dpb, wb, sb, tmp, stg4, stg8, z8v, sem0, sem1, sem2):
    c = lax.axis_index("c")
    t = lax.axis_index("s")

    pltpu.sync_copy(zrow_hbm, tmp)
    for qi in range(2 * NSLICE // NTILE + 1):
        q = t + qi * NTILE

        @pl.when(q < 2 * NSLICE)
        def _():
            r = q * SR
            pltpu.sync_copy(asrc_hbm.at[pl.ds(r, SR)], stg4)
            pltpu.sync_copy(stg4, asrc_sh.at[pl.ds(r, SR)])

    for qi in range(4):
        q = t + qi * NTILE

        @pl.when(q < NSLICE)
        def _():
            r = q * SR
            pltpu.sync_copy(dpk_hbm.at[c, pl.ds(r, SR)], stg8)
            pltpu.sync_copy(stg8, dpk_sh.at[pl.ds(r, SR)])
            pltpu.sync_copy(tmp, acc_sh.at[pl.ds(r, SR)])

    @pl.when(t == 0)
    def _():
        pltpu.sync_copy(tmp.at[pl.ds(0, 8)], acc_sh.at[pl.ds(N, 8)])
        pltpu.sync_copy(z8_hbm, z8v)
        pltpu.sync_copy(z8v, dpk_sh.at[pl.ds(N, 8)])

    lanes = lax.iota(jnp.int32, 16)
    zv = jnp.zeros((16,), jnp.float32)
    for g in range(K // 16):
        for pc in range(HD + H, ACCW):
            plsc.store_scatter(sb, [lanes + g * 16,
                                    jnp.full((16,), pc, jnp.int32)], zv)

    plsc.subcore_barrier()

    quad = lanes // 4
    hmod = lanes - quad * 4

    def blk_body(bi, carry):
        rowb = c * (E_PAD // K) + t * (EPT // K) + bi * NBLK
        b1 = pltpu.async_copy(src_hbm.at[pl.ds(rowb, NBLK)], srcb, sem0)
        b2 = pltpu.async_copy(dst_hbm.at[pl.ds(rowb, NBLK)], dstb, sem1)
        b3 = pltpu.async_copy(ew_hbm.at[pl.ds(rowb, NBLK)], ewb, sem2)
        b1.wait()
        b2.wait()
        b3.wait()

        def chunk_body(ci, carry2):
            srcv = srcb.at[ci]
            dstv = dstb.at[ci]
            d1 = pltpu.async_copy(xs_hbm.at[srcv], xsb, sem0)
            d2 = pltpu.async_copy(asrc_sh.at[srcv], asb, sem1)
            d3 = pltpu.async_copy(dpk_sh.at[dstv], dpb, sem2)
            d1.wait()
            d2.wait()
            d3.wait()
            civ = jnp.full((16,), ci, jnp.int32)

            for j in range(K * H // 16):
                ev = quad + j * 4
                a_s = plsc.load_gather(asb, [ev, hmod])
                a_d = plsc.load_gather(dpb, [ev, hmod])
                ubv = plsc.load_gather(dpb, [ev, hmod + H])
                eww = plsc.load_gather(ewb, [civ, ev])
                a = a_s + a_d
                a = jnp.maximum(a, NEG * a)
                ex = jnp.exp(a - ubv)
                plsc.store_scatter(sb, [ev, hmod + HD], ex)
                plsc.store_scatter(wb, [ev, hmod], ex * eww)

            def grp(g, carry3):
                evg = lanes + g * 16
                for h in range(H):
                    wv = plsc.load_gather(
                        wb, [evg, jnp.full((16,), h, jnp.int32)])
                    for d in range(D):
                        col = jnp.full((16,), h * D + d, jnp.int32)
                        xv = plsc.load_gather(xsb, [evg, col])
                        plsc.store_scatter(sb, [evg, col], xv * wv)
                return carry3

            lax.fori_loop(0, K // 16, grp, 0)

            pltpu.sync_copy(sb, acc_sh.at[dstv], add=True)
            return carry2

        lax.fori_loop(0, NBLK, chunk_body, 0)
        return carry

    lax.fori_loop(0, NBLOCK, blk_body, 0)

    plsc.subcore_barrier()

    for qi in range(4):
        q = t + qi * NTILE

        @pl.when(q < NSLICE)
        def _():
            r = q * SR
            pltpu.sync_copy(acc_sh.at[pl.ds(r, SR)], tmp)
            pltpu.sync_copy(tmp, out_hbm.at[c, pl.ds(r, SR)])


def _final_body(acc_ref, xs_ref, asrc_ref, dpk_ref, gatb_ref, masks_ref,
                scales_ref, embW_ref, embb_ref, R_ref, emb_ref, ns_ref):
    ns = jax.nn.softmax(scales_ref[...], axis=-1)
    im = jax.nn.softmax(masks_ref[...], axis=-1)
    R = R_ref[...]
    accn = jnp.zeros((TR, HD), jnp.float32)
    for i in range(M):
        msg = acc_ref[i, :, 0:HD]
        dsum = acc_ref[i, :, HD:HD + H]
        s = asrc_ref[i] + dpk_ref[i, :, 0:H]
        ub = dpk_ref[i, :, H:2 * H]
        asl = jnp.exp(jnp.maximum(s, NEG * s) - ub)
        rec = 1.0 / (dsum + asl + 1e-16)
        asl64 = jnp.dot(asl, R, preferred_element_type=jnp.float32)
        rec64 = jnp.dot(rec, R, preferred_element_type=jnp.float32)
        conv = (msg + asl64 * xs_ref[i]) * rec64 + gatb_ref[i]
        accn = accn + conv * (im[:, i:i + 1] * ns[:, i:i + 1])
    emb_ref[...] = jnp.dot(accn, embW_ref[...],
                           preferred_element_type=jnp.float32) + embb_ref[...]
    ns_ref[...] = ns


def _final(acc, xs_all, asrc_all, dpk_all, gatb, masks, scales, embW, embb, R):
    return pl.pallas_call(
        _final_body,
        grid=(N // TR,),
        in_specs=[
            pl.BlockSpec((M, TR, ACCW), lambda r: (0, r, 0)),
            pl.BlockSpec((M, TR, HD), lambda r: (0, r, 0)),
            pl.BlockSpec((M, TR, H), lambda r: (0, r, 0)),
            pl.BlockSpec((M, TR, 2 * H), lambda r: (0, r, 0)),
            pl.BlockSpec((M, 1, HD), lambda r: (0, 0, 0)),
            pl.BlockSpec((TR, M), lambda r: (r, 0)),
            pl.BlockSpec((1, M), lambda r: (0, 0)),
            pl.BlockSpec((HD, EMB), lambda r: (0, 0)),
            pl.BlockSpec((1, EMB), lambda r: (0, 0)),
            pl.BlockSpec((H, HD), lambda r: (0, 0)),
        ],
        out_specs=(
            pl.BlockSpec((TR, EMB), lambda r: (r, 0)),
            pl.BlockSpec((1, M), lambda r: (0, 0)),
        ),
        out_shape=(
            jax.ShapeDtypeStruct((N, EMB), jnp.float32),
            jax.ShapeDtypeStruct((1, M), jnp.float32),
        ),
    )(acc, xs_all, asrc_all, dpk_all, gatb, masks, scales, embW, embb, R)


def _dot_body(a_ref, b_ref, o_ref):
    o_ref[...] = lax.dot_general(
        a_ref[...], b_ref[...], (((1,), (1,)), ((), ())),
        preferred_element_type=jnp.float32)


def _dot(emb):
    return pl.pallas_call(
        _dot_body,
        grid=(N // TM,),
        in_specs=[
            pl.BlockSpec((TM, EMB), lambda i: (i, 0)),
            pl.BlockSpec((N, EMB), lambda i: (0, 0)),
        ],
        out_specs=pl.BlockSpec((TM, N), lambda i: (i, 0)),
        out_shape=jax.ShapeDtypeStruct((N, N), jnp.float32),
    )(emb, emb)


def kernel(features, masks, edge_index0, edge_weight0, edge_index1, edge_weight1,
           pre_W0, pre_b0, pre_W1, pre_b1,
           lin_src_W0, lin_dst_W0, att_src0, att_dst0, gat_b0,
           lin_src_W1, lin_dst_W1, att_src1, att_dst1, gat_b1,
           scales, emb_W, emb_b):
    f32 = jnp.float32
    i32 = jnp.int32
    preW = jnp.stack([pre_W0, pre_W1])
    preb = jnp.stack([pre_b0.reshape(1, HD), pre_b1.reshape(1, HD)])
    linS = jnp.stack([lin_src_W0, lin_src_W1])
    linD = jnp.stack([lin_dst_W0, lin_dst_W1])
    R = jnp.kron(jnp.eye(H, dtype=f32), jnp.ones((1, D), f32))
    AS = jnp.stack([R.T * att_src0.reshape(HD, 1),
                    R.T * att_src1.reshape(HD, 1)])
    AD = jnp.stack([R.T * att_dst0.reshape(HD, 1),
                    R.T * att_dst1.reshape(HD, 1)])
    gatb = jnp.stack([gat_b0.reshape(1, HD), gat_b1.reshape(1, HD)])

    xs_all, asrc_all, dpk_all = _prep(features, preW, preb, linS, linD, AS, AD)

    pad = E_PAD - E
    src_flat = jnp.concatenate([
        edge_index0[0], jnp.zeros((pad,), i32),
        edge_index1[0] + N, jnp.full((pad,), N, i32)])
    dst_flat = jnp.concatenate([
        edge_index0[1], jnp.full((pad,), N, i32),
        edge_index1[1], jnp.full((pad,), N, i32)])
    ew_flat = jnp.concatenate([
        edge_weight0, jnp.zeros((pad,), f32),
        edge_weight1, jnp.zeros((pad,), f32)])
    zrow = jnp.zeros((SR, ACCW), f32)
    z8 = jnp.zeros((8, 2 * H), f32)

    asrc8 = jnp.concatenate(
        [asrc_all.reshape(M * N, H), jnp.zeros((M * N, H), f32)], axis=1)
    acc = _sc_edge(src_flat.reshape(-1, K), dst_flat.reshape(-1, K),
                   ew_flat.reshape(-1, K), xs_all.reshape(M * N, HD),
                   asrc8, dpk_all, zrow, z8)

    emb, ns = _final(acc, xs_all, asrc_all, dpk_all, gatb, masks,
                     scales.reshape(1, M), emb_W, emb_b.reshape(1, EMB), R)
    dot = _dot(emb)
    return dot, emb, ns
